# baseline (speedup 1.0000x reference)
import functools

import jax
import jax.numpy as jnp
from jax.experimental import pallas as pl
from jax.experimental.pallas import tpu as pltpu

D_MODEL_ = 768
D_FF_ = 3072
N_EXP_ = 8
N_TOK_ = 2048
LANES_ = 128


def _gelu_exact(h):
    return h * 0.5 * (1.0 + jax.lax.erf(h * jnp.float32(0.7071067811865476)))


def _router_combine(xt, wr_pad, br_pad):
    lane = jax.lax.broadcasted_iota(jnp.int32, (TOK_BLK_, LANES_), 1)
    valid = lane < N_EXP_
    logits = jax.lax.dot_general(
        xt, wr_pad, (((1,), (0,)), ((), ())),
        preferred_element_type=jnp.float32) + br_pad
    neg = jnp.float32(-1e30)
    logits = jnp.where(valid, logits, neg)
    m = jnp.max(logits, axis=1, keepdims=True)
    e = jnp.where(valid, jnp.exp(logits - m), 0.0)
    s = jnp.sum(e, axis=1, keepdims=True)
    probs = e / s
    m1 = jnp.max(probs, axis=1, keepdims=True)
    i1 = jnp.min(jnp.where(probs == m1, lane, N_EXP_), axis=1, keepdims=True)
    sel1 = lane == i1
    probs2 = jnp.where(sel1, -1.0, probs)
    m2 = jnp.max(probs2, axis=1, keepdims=True)
    i2 = jnp.min(jnp.where(probs2 == m2, lane, N_EXP_), axis=1, keepdims=True)
    sel2 = lane == i2
    tot = jnp.clip(m1 + m2, 1e-9, None)
    return (jnp.where(sel1, m1, 0.0) + jnp.where(sel2, m2, 0.0)) / tot

TOK_BLK_ = 512
N_TBLK_ = N_TOK_ // TOK_BLK_


def _moe_dense_kernel(xt_ref, wr_ref, br_ref, w1_ref, b1_ref, w2_ref, b2_ref,
                      out_ref, comb_ref):
    e = pl.program_id(1)

    @pl.when(e == 0)
    def _():
        comb_ref[...] = _router_combine(xt_ref[...], wr_ref[...], br_ref[...])
        out_ref[...] = jnp.zeros_like(out_ref)

    lane = jax.lax.broadcasted_iota(jnp.int32, (TOK_BLK_, LANES_), 1)
    w_col = jnp.sum(jnp.where(lane == e, comb_ref[...], 0.0), axis=1,
                    keepdims=True)
    h = jax.lax.dot_general(
        xt_ref[...].astype(jnp.bfloat16), w1_ref[0], (((1,), (0,)), ((), ())),
        preferred_element_type=jnp.float32) + b1_ref[0]
    h = _gelu_exact(h)
    y = jax.lax.dot_general(
        h.astype(jnp.bfloat16), w2_ref[0], (((1,), (0,)), ((), ())),
        preferred_element_type=jnp.float32) + b2_ref[0]
    out_ref[...] += w_col * y


@jax.jit
def kernel(x, Wr, br, W1, b1, W2, b2):
    orig_shape = x.shape
    xt = x.reshape(-1, D_MODEL_)
    wr_pad = jnp.zeros((D_MODEL_, LANES_), jnp.float32).at[:, :N_EXP_].set(Wr)
    br_pad = jnp.zeros((LANES_,), jnp.float32).at[:N_EXP_].set(br)

    out = pl.pallas_call(
        _moe_dense_kernel,
        grid=(N_TBLK_, N_EXP_),
        in_specs=[
            pl.BlockSpec((TOK_BLK_, D_MODEL_), lambda t, e: (t, 0)),
            pl.BlockSpec((D_MODEL_, LANES_), lambda t, e: (0, 0)),
            pl.BlockSpec((LANES_,), lambda t, e: (0,)),
            pl.BlockSpec((1, D_MODEL_, D_FF_), lambda t, e: (e, 0, 0)),
            pl.BlockSpec((1, 1, D_FF_), lambda t, e: (e, 0, 0)),
            pl.BlockSpec((1, D_FF_, D_MODEL_), lambda t, e: (e, 0, 0)),
            pl.BlockSpec((1, 1, D_MODEL_), lambda t, e: (e, 0, 0)),
        ],
        out_specs=pl.BlockSpec((TOK_BLK_, D_MODEL_), lambda t, e: (t, 0)),
        out_shape=jax.ShapeDtypeStruct((N_TOK_, D_MODEL_), jnp.float32),
        scratch_shapes=[pltpu.VMEM((TOK_BLK_, LANES_), jnp.float32)],
    )(xt, wr_pad, br_pad, W1.astype(jnp.bfloat16),
      b1.reshape(N_EXP_, 1, D_FF_), W2.astype(jnp.bfloat16),
      b2.reshape(N_EXP_, 1, D_MODEL_))
    return out.reshape(orig_shape)


# speedup vs baseline: 1.5775x; 1.5775x over previous
import functools

import jax
import jax.numpy as jnp
from jax import lax
from jax.experimental import pallas as pl
from jax.experimental.pallas import tpu as pltpu
from jax.experimental.pallas import tpu_sc as plsc

D_MODEL_ = 768
D_FF_ = 3072
N_EXP_ = 8
N_TOK_ = 2048
LANES_ = 128
N_ASSIGN_ = 2 * N_TOK_
BLK_ = 256
GMAX_ = 24
S_PAD_ = GMAX_ * BLK_
CHUNK_ = 512
N_CHUNK_ = N_ASSIGN_ // CHUNK_
SQRT1_2_ = 0.7071067811865476


def _gelu_exact(h):
    return h * 0.5 * (1.0 + lax.erf(h * jnp.float32(SQRT1_2_)))


def _route_meta_kernel(xt_ref, wr_ref, br_ref,
                       dest_ref, w_ref, eblk_ref, valid_ref,
                       m_ref, r_ref):
    lane = lax.broadcasted_iota(jnp.int32, (N_TOK_, LANES_), 1)
    evalid = lane < N_EXP_
    logits = lax.dot_general(
        xt_ref[...], wr_ref[...], (((1,), (0,)), ((), ())),
        preferred_element_type=jnp.float32) + br_ref[...]
    logits = jnp.where(evalid, logits, jnp.float32(-1e30))
    mx = jnp.max(logits, axis=1, keepdims=True)
    ex = jnp.where(evalid, jnp.exp(logits - mx), 0.0)
    probs = ex / jnp.sum(ex, axis=1, keepdims=True)
    m1 = jnp.max(probs, axis=1, keepdims=True)
    i1 = jnp.min(jnp.where(probs == m1, lane, N_EXP_), axis=1, keepdims=True)
    sel1 = lane == i1
    probs2 = jnp.where(sel1, -1.0, probs)
    m2 = jnp.max(probs2, axis=1, keepdims=True)
    i2 = jnp.min(jnp.where(probs2 == m2, lane, N_EXP_), axis=1, keepdims=True)
    sel2 = lane == i2
    tot = jnp.clip(m1 + m2, 1e-9, None)
    w_ref[...] = (jnp.where(lane == 0, m1, 0.0)
                  + jnp.where(lane == 1, m2, 0.0)) / tot

    m_ref[:N_TOK_] = sel1.astype(jnp.float32)
    m_ref[N_TOK_:] = sel2.astype(jnp.float32)

    row = lax.broadcasted_iota(jnp.int32, (CHUNK_, CHUNK_), 0)
    col = lax.broadcasted_iota(jnp.int32, (CHUNK_, CHUNK_), 1)
    tril_strict = (col < row).astype(jnp.float32)

    def body(c, carry):
        off = pl.multiple_of(c * CHUNK_, CHUNK_)
        mc = m_ref[pl.ds(off, CHUNK_), :]
        rank_c = lax.dot_general(
            tril_strict, mc, (((1,), (0,)), ((), ())),
            preferred_element_type=jnp.float32) + carry
        r_ref[pl.ds(off, CHUNK_), :] = jnp.sum(rank_c * mc, axis=1,
                                               keepdims=True)
        return carry + jnp.sum(mc, axis=0, keepdims=True)

    counts = lax.fori_loop(0, N_CHUNK_, body, jnp.zeros((1, LANES_),
                                                        jnp.float32))

    nblk = jnp.floor((counts + (BLK_ - 1)) * (1.0 / BLK_))
    rowi = lax.broadcasted_iota(jnp.int32, (LANES_, LANES_), 0)
    coli = lax.broadcasted_iota(jnp.int32, (LANES_, LANES_), 1)
    triu_strict = (rowi < coli).astype(jnp.float32)
    blkoff = lax.dot_general(nblk, triu_strict, (((1,), (0,)), ((), ())),
                             preferred_element_type=jnp.float32)
    off_pad = blkoff * jnp.float32(BLK_)

    def body2(c, _):
        off = pl.multiple_of(c * CHUNK_, CHUNK_)
        mc = m_ref[pl.ds(off, CHUNK_), :]
        base = jnp.sum(mc * off_pad, axis=1, keepdims=True)
        dest_ref[pl.ds(off, CHUNK_), :] = (
            r_ref[pl.ds(off, CHUNK_), :] + base).astype(jnp.int32)
        return 0

    lax.fori_loop(0, N_CHUNK_, body2, 0)

    ones = jnp.ones((LANES_, LANES_), jnp.float32)
    onehot8 = jnp.logical_and(rowi == coli, rowi < N_EXP_).astype(jnp.float32)
    off_rows = onehot8 * blkoff
    off_bcast = lax.dot_general(off_rows, ones, (((1,), (0,)), ((), ())),
                                preferred_element_type=jnp.float32)
    g_row = lax.broadcasted_iota(jnp.int32, (N_EXP_, LANES_), 1
                                 ).astype(jnp.float32)
    ge = (g_row >= off_bcast[:N_EXP_]).astype(jnp.float32)
    eblk_ref[...] = (jnp.sum(ge, axis=0, keepdims=True) - 1.0
                     ).astype(jnp.int32)
    emask = lax.broadcasted_iota(jnp.int32, (1, LANES_), 1) < N_EXP_
    total = lax.dot_general(jnp.where(emask, nblk, 0.0), ones,
                            (((1,), (0,)), ((), ())),
                            preferred_element_type=jnp.float32)
    gl = lax.broadcasted_iota(jnp.int32, (1, LANES_), 1).astype(jnp.float32)
    valid_ref[...] = (gl < total).astype(jnp.int32)


def _gmm_kernel(eblk_ref, valid_ref, xs_ref, w1_ref, b1_ref, w2_ref, b2_ref,
                ys_ref):
    g = pl.program_id(0)

    @pl.when(valid_ref[g] == 1)
    def _():
        h = lax.dot_general(
            xs_ref[...], w1_ref[0], (((1,), (0,)), ((), ())),
            preferred_element_type=jnp.float32) + b1_ref[0]
        h = _gelu_exact(h)
        ys_ref[...] = lax.dot_general(
            h, w2_ref[0], (((1,), (0,)), ((), ())),
            preferred_element_type=jnp.float32) + b2_ref[0]


def _combine_kernel(yp_ref, w_ref, out_ref):
    lane = lax.broadcasted_iota(jnp.int32, (N_TOK_, LANES_), 1)
    w0 = jnp.sum(jnp.where(lane == 0, w_ref[...], 0.0), axis=1, keepdims=True)
    w1 = jnp.sum(jnp.where(lane == 1, w_ref[...], 0.0), axis=1, keepdims=True)
    out_ref[...] = w0 * yp_ref[:N_TOK_] + w1 * yp_ref[N_TOK_:]


_NC_ = 2
_NS_ = 16
_NW_ = _NC_ * _NS_
_APW_ = N_ASSIGN_ // _NW_


def _sc_mesh():
    return plsc.VectorSubcoreMesh(core_axis_name="c", subcore_axis_name="s",
                                  num_cores=_NC_, num_subcores=_NS_)


def _scatter_x_body(xt_hbm, dest_hbm, xs_hbm, idx_v, rows_v, sem):
    wid = lax.axis_index("s") * _NC_ + lax.axis_index("c")
    base = wid * _APW_
    src = lax.rem(base, N_TOK_)
    pltpu.sync_copy(dest_hbm.at[pl.ds(base, _APW_)], idx_v)
    pltpu.sync_copy(xt_hbm.at[pl.ds(src, _APW_)], rows_v)
    pltpu.async_copy(rows_v, xs_hbm.at[idx_v], sem).wait()


def _gather_y_body(ys_hbm, dest_hbm, yp_hbm, idx_v, rows_v, sem):
    wid = lax.axis_index("s") * _NC_ + lax.axis_index("c")
    base = wid * _APW_
    pltpu.sync_copy(dest_hbm.at[pl.ds(base, _APW_)], idx_v)
    pltpu.async_copy(ys_hbm.at[idx_v], rows_v, sem).wait()
    pltpu.sync_copy(rows_v, yp_hbm.at[pl.ds(base, _APW_)])


def _sc_scatter_x(xt, dest):
    return pl.kernel(
        _scatter_x_body,
        out_type=jax.ShapeDtypeStruct((S_PAD_, D_MODEL_), jnp.float32),
        mesh=_sc_mesh(),
        scratch_types=[
            pltpu.VMEM((_APW_,), jnp.int32),
            pltpu.VMEM((_APW_, D_MODEL_), jnp.float32),
            pltpu.SemaphoreType.DMA,
        ],
    )(xt, dest)


def _sc_gather_y(ys, dest):
    return pl.kernel(
        _gather_y_body,
        out_type=jax.ShapeDtypeStruct((N_ASSIGN_, D_MODEL_), jnp.float32),
        mesh=_sc_mesh(),
        scratch_types=[
            pltpu.VMEM((_APW_,), jnp.int32),
            pltpu.VMEM((_APW_, D_MODEL_), jnp.float32),
            pltpu.SemaphoreType.DMA,
        ],
    )(ys, dest)


@jax.jit
def kernel(x, Wr, br, W1, b1, W2, b2):
    orig_shape = x.shape
    xt = x.reshape(-1, D_MODEL_)
    wr_pad = jnp.zeros((D_MODEL_, LANES_), jnp.float32).at[:, :N_EXP_].set(Wr)
    br_pad = jnp.zeros((LANES_,), jnp.float32).at[:N_EXP_].set(br)

    dest2d, w_pair, eblk2d, valid2d = pl.pallas_call(
        _route_meta_kernel,
        in_specs=[
            pl.BlockSpec((N_TOK_, D_MODEL_), lambda: (0, 0)),
            pl.BlockSpec((D_MODEL_, LANES_), lambda: (0, 0)),
            pl.BlockSpec((LANES_,), lambda: (0,)),
        ],
        out_specs=[
            pl.BlockSpec((N_ASSIGN_, 1), lambda: (0, 0)),
            pl.BlockSpec((N_TOK_, LANES_), lambda: (0, 0)),
            pl.BlockSpec((1, LANES_), lambda: (0, 0)),
            pl.BlockSpec((1, LANES_), lambda: (0, 0)),
        ],
        out_shape=[
            jax.ShapeDtypeStruct((N_ASSIGN_, 1), jnp.int32),
            jax.ShapeDtypeStruct((N_TOK_, LANES_), jnp.float32),
            jax.ShapeDtypeStruct((1, LANES_), jnp.int32),
            jax.ShapeDtypeStruct((1, LANES_), jnp.int32),
        ],
        scratch_shapes=[
            pltpu.VMEM((N_ASSIGN_, LANES_), jnp.float32),
            pltpu.VMEM((N_ASSIGN_, 1), jnp.float32),
        ],
    )(xt, wr_pad, br_pad)

    dest = dest2d.reshape(N_ASSIGN_)
    eblk = eblk2d.reshape(LANES_)[:GMAX_]
    valid = valid2d.reshape(LANES_)[:GMAX_]

    xs = _sc_scatter_x(xt, dest)

    ys = pl.pallas_call(
        _gmm_kernel,
        grid_spec=pltpu.PrefetchScalarGridSpec(
            num_scalar_prefetch=2,
            grid=(GMAX_,),
            in_specs=[
                pl.BlockSpec((BLK_, D_MODEL_), lambda g, eb, vd: (g, 0)),
                pl.BlockSpec((1, D_MODEL_, D_FF_),
                             lambda g, eb, vd: (eb[g], 0, 0)),
                pl.BlockSpec((1, 1, D_FF_), lambda g, eb, vd: (eb[g], 0, 0)),
                pl.BlockSpec((1, D_FF_, D_MODEL_),
                             lambda g, eb, vd: (eb[g], 0, 0)),
                pl.BlockSpec((1, 1, D_MODEL_),
                             lambda g, eb, vd: (eb[g], 0, 0)),
            ],
            out_specs=pl.BlockSpec((BLK_, D_MODEL_), lambda g, eb, vd: (g, 0)),
        ),
        out_shape=jax.ShapeDtypeStruct((S_PAD_, D_MODEL_), jnp.float32),
    )(eblk, valid, xs, W1, b1.reshape(N_EXP_, 1, D_FF_), W2,
      b2.reshape(N_EXP_, 1, D_MODEL_))

    yp = _sc_gather_y(ys, dest)

    out = pl.pallas_call(
        _combine_kernel,
        in_specs=[
            pl.BlockSpec((N_ASSIGN_, D_MODEL_), lambda: (0, 0)),
            pl.BlockSpec((N_TOK_, LANES_), lambda: (0, 0)),
        ],
        out_specs=pl.BlockSpec((N_TOK_, D_MODEL_), lambda: (0, 0)),
        out_shape=jax.ShapeDtypeStruct((N_TOK_, D_MODEL_), jnp.float32),
    )(yp, w_pair)
    return out.reshape(orig_shape)


# speedup vs baseline: 1.5859x; 1.0053x over previous
import functools

import jax
import jax.numpy as jnp
from jax import lax
from jax.experimental import pallas as pl
from jax.experimental.pallas import tpu as pltpu
from jax.experimental.pallas import tpu_sc as plsc

D_MODEL_ = 768
D_FF_ = 3072
N_EXP_ = 8
N_TOK_ = 2048
LANES_ = 128
N_ASSIGN_ = 2 * N_TOK_
BLK_ = 256
GMAX_ = 24
S_PAD_ = GMAX_ * BLK_
CHUNK_ = 1024
N_CHUNK_ = N_ASSIGN_ // CHUNK_
SQRT1_2_ = 0.7071067811865476


def _gelu_exact(h):
    return h * 0.5 * (1.0 + lax.erf(h * jnp.float32(SQRT1_2_)))


def _route_meta_kernel(xt_ref, wr_ref, br_ref,
                       dest_ref, w_ref, eblk_ref, valid_ref, m_ref):
    lane = lax.broadcasted_iota(jnp.int32, (N_TOK_, LANES_), 1)
    evalid = lane < N_EXP_
    logits = lax.dot_general(
        xt_ref[...], wr_ref[...], (((1,), (0,)), ((), ())),
        preferred_element_type=jnp.float32) + br_ref[...]
    logits = jnp.where(evalid, logits, jnp.float32(-1e30))
    mx = jnp.max(logits, axis=1, keepdims=True)
    ex = jnp.where(evalid, jnp.exp(logits - mx), 0.0)
    probs = ex / jnp.sum(ex, axis=1, keepdims=True)
    m1 = jnp.max(probs, axis=1, keepdims=True)
    i1 = jnp.min(jnp.where(probs == m1, lane, N_EXP_), axis=1, keepdims=True)
    sel1 = lane == i1
    probs2 = jnp.where(sel1, -1.0, probs)
    m2 = jnp.max(probs2, axis=1, keepdims=True)
    i2 = jnp.min(jnp.where(probs2 == m2, lane, N_EXP_), axis=1, keepdims=True)
    sel2 = lane == i2
    tot = jnp.clip(m1 + m2, 1e-9, None)
    w_ref[...] = (jnp.where(lane == 0, m1, 0.0)
                  + jnp.where(lane == 1, m2, 0.0)) / tot

    sel1f = sel1.astype(jnp.float32)
    sel2f = sel2.astype(jnp.float32)
    m_ref[:N_TOK_] = sel1f
    m_ref[N_TOK_:] = sel2f
    counts = (jnp.sum(sel1f, axis=0, keepdims=True)
              + jnp.sum(sel2f, axis=0, keepdims=True))

    nblk = jnp.floor((counts + (BLK_ - 1)) * (1.0 / BLK_))
    rowi = lax.broadcasted_iota(jnp.int32, (LANES_, LANES_), 0)
    coli = lax.broadcasted_iota(jnp.int32, (LANES_, LANES_), 1)
    triu_strict = (rowi < coli).astype(jnp.float32)
    blkoff = lax.dot_general(nblk, triu_strict, (((1,), (0,)), ((), ())),
                             preferred_element_type=jnp.float32)
    off_pad = blkoff * jnp.float32(BLK_)

    row = lax.broadcasted_iota(jnp.int32, (CHUNK_, CHUNK_), 0)
    col = lax.broadcasted_iota(jnp.int32, (CHUNK_, CHUNK_), 1)
    tril_strict = (col < row).astype(jnp.float32)

    def body(c, carry):
        off = pl.multiple_of(c * CHUNK_, CHUNK_)
        mc = m_ref[pl.ds(off, CHUNK_), :]
        rank_c = lax.dot_general(
            tril_strict, mc, (((1,), (0,)), ((), ())),
            preferred_element_type=jnp.float32) + carry
        dest_ref[pl.ds(off, CHUNK_), :] = jnp.sum(
            (rank_c + off_pad) * mc, axis=1, keepdims=True).astype(jnp.int32)
        return carry + jnp.sum(mc, axis=0, keepdims=True)

    lax.fori_loop(0, N_CHUNK_, body, jnp.zeros((1, LANES_), jnp.float32))

    ones = jnp.ones((LANES_, LANES_), jnp.float32)
    onehot8 = jnp.logical_and(rowi == coli, rowi < N_EXP_).astype(jnp.float32)
    off_rows = onehot8 * blkoff
    off_bcast = lax.dot_general(off_rows, ones, (((1,), (0,)), ((), ())),
                                preferred_element_type=jnp.float32)
    g_row = lax.broadcasted_iota(jnp.int32, (N_EXP_, LANES_), 1
                                 ).astype(jnp.float32)
    ge = (g_row >= off_bcast[:N_EXP_]).astype(jnp.float32)
    eblk_ref[...] = (jnp.sum(ge, axis=0, keepdims=True) - 1.0
                     ).astype(jnp.int32)
    emask = lax.broadcasted_iota(jnp.int32, (1, LANES_), 1) < N_EXP_
    total = lax.dot_general(jnp.where(emask, nblk, 0.0), ones,
                            (((1,), (0,)), ((), ())),
                            preferred_element_type=jnp.float32)
    gl = lax.broadcasted_iota(jnp.int32, (1, LANES_), 1).astype(jnp.float32)
    valid_ref[...] = (gl < total).astype(jnp.int32)


def _gmm_kernel(eblk_ref, valid_ref, xs_ref, w1_ref, b1_ref, w2_ref, b2_ref,
                ys_ref):
    g = pl.program_id(0)

    @pl.when(valid_ref[g] == 1)
    def _():
        h = lax.dot_general(
            xs_ref[...], w1_ref[0], (((1,), (0,)), ((), ())),
            preferred_element_type=jnp.float32) + b1_ref[0]
        h = _gelu_exact(h)
        ys_ref[...] = lax.dot_general(
            h, w2_ref[0], (((1,), (0,)), ((), ())),
            preferred_element_type=jnp.float32) + b2_ref[0]


def _combine_kernel(yp_ref, w_ref, out_ref):
    lane = lax.broadcasted_iota(jnp.int32, (N_TOK_, LANES_), 1)
    w0 = jnp.sum(jnp.where(lane == 0, w_ref[...], 0.0), axis=1, keepdims=True)
    w1 = jnp.sum(jnp.where(lane == 1, w_ref[...], 0.0), axis=1, keepdims=True)
    out_ref[...] = w0 * yp_ref[:N_TOK_] + w1 * yp_ref[N_TOK_:]


_NC_ = 2
_NS_ = 16
_NW_ = _NC_ * _NS_
_APW_ = N_ASSIGN_ // _NW_


def _sc_mesh():
    return plsc.VectorSubcoreMesh(core_axis_name="c", subcore_axis_name="s",
                                  num_cores=_NC_, num_subcores=_NS_)


def _scatter_x_body(xt_hbm, dest_hbm, xs_hbm, idx_v, rows_v, sem):
    wid = lax.axis_index("s") * _NC_ + lax.axis_index("c")
    base = wid * _APW_
    src = lax.rem(base, N_TOK_)
    pltpu.sync_copy(dest_hbm.at[pl.ds(base, _APW_)], idx_v)
    pltpu.sync_copy(xt_hbm.at[pl.ds(src, _APW_)], rows_v)
    pltpu.async_copy(rows_v, xs_hbm.at[idx_v], sem).wait()


def _gather_y_body(ys_hbm, dest_hbm, yp_hbm, idx_v, rows_v, sem):
    wid = lax.axis_index("s") * _NC_ + lax.axis_index("c")
    base = wid * _APW_
    pltpu.sync_copy(dest_hbm.at[pl.ds(base, _APW_)], idx_v)
    pltpu.async_copy(ys_hbm.at[idx_v], rows_v, sem).wait()
    pltpu.sync_copy(rows_v, yp_hbm.at[pl.ds(base, _APW_)])


def _sc_scatter_x(xt, dest):
    return pl.kernel(
        _scatter_x_body,
        out_type=jax.ShapeDtypeStruct((S_PAD_, D_MODEL_), jnp.float32),
        mesh=_sc_mesh(),
        scratch_types=[
            pltpu.VMEM((_APW_,), jnp.int32),
            pltpu.VMEM((_APW_, D_MODEL_), jnp.float32),
            pltpu.SemaphoreType.DMA,
        ],
    )(xt, dest)


def _sc_gather_y(ys, dest):
    return pl.kernel(
        _gather_y_body,
        out_type=jax.ShapeDtypeStruct((N_ASSIGN_, D_MODEL_), jnp.float32),
        mesh=_sc_mesh(),
        scratch_types=[
            pltpu.VMEM((_APW_,), jnp.int32),
            pltpu.VMEM((_APW_, D_MODEL_), jnp.float32),
            pltpu.SemaphoreType.DMA,
        ],
    )(ys, dest)


@jax.jit
def kernel(x, Wr, br, W1, b1, W2, b2):
    orig_shape = x.shape
    xt = x.reshape(-1, D_MODEL_)
    wr_pad = jnp.zeros((D_MODEL_, LANES_), jnp.float32).at[:, :N_EXP_].set(Wr)
    br_pad = jnp.zeros((LANES_,), jnp.float32).at[:N_EXP_].set(br)

    dest2d, w_pair, eblk2d, valid2d = pl.pallas_call(
        _route_meta_kernel,
        in_specs=[
            pl.BlockSpec((N_TOK_, D_MODEL_), lambda: (0, 0)),
            pl.BlockSpec((D_MODEL_, LANES_), lambda: (0, 0)),
            pl.BlockSpec((LANES_,), lambda: (0,)),
        ],
        out_specs=[
            pl.BlockSpec((N_ASSIGN_, 1), lambda: (0, 0)),
            pl.BlockSpec((N_TOK_, LANES_), lambda: (0, 0)),
            pl.BlockSpec((1, LANES_), lambda: (0, 0)),
            pl.BlockSpec((1, LANES_), lambda: (0, 0)),
        ],
        out_shape=[
            jax.ShapeDtypeStruct((N_ASSIGN_, 1), jnp.int32),
            jax.ShapeDtypeStruct((N_TOK_, LANES_), jnp.float32),
            jax.ShapeDtypeStruct((1, LANES_), jnp.int32),
            jax.ShapeDtypeStruct((1, LANES_), jnp.int32),
        ],
        scratch_shapes=[
            pltpu.VMEM((N_ASSIGN_, LANES_), jnp.float32),
        ],
    )(xt, wr_pad, br_pad)

    dest = dest2d.reshape(N_ASSIGN_)
    eblk = eblk2d.reshape(LANES_)[:GMAX_]
    valid = valid2d.reshape(LANES_)[:GMAX_]

    xs = _sc_scatter_x(xt, dest)

    ys = pl.pallas_call(
        _gmm_kernel,
        grid_spec=pltpu.PrefetchScalarGridSpec(
            num_scalar_prefetch=2,
            grid=(GMAX_,),
            in_specs=[
                pl.BlockSpec((BLK_, D_MODEL_), lambda g, eb, vd: (g, 0)),
                pl.BlockSpec((1, D_MODEL_, D_FF_),
                             lambda g, eb, vd: (eb[g], 0, 0)),
                pl.BlockSpec((1, 1, D_FF_), lambda g, eb, vd: (eb[g], 0, 0)),
                pl.BlockSpec((1, D_FF_, D_MODEL_),
                             lambda g, eb, vd: (eb[g], 0, 0)),
                pl.BlockSpec((1, 1, D_MODEL_),
                             lambda g, eb, vd: (eb[g], 0, 0)),
            ],
            out_specs=pl.BlockSpec((BLK_, D_MODEL_), lambda g, eb, vd: (g, 0)),
        ),
        out_shape=jax.ShapeDtypeStruct((S_PAD_, D_MODEL_), jnp.float32),
    )(eblk, valid, xs, W1, b1.reshape(N_EXP_, 1, D_FF_), W2,
      b2.reshape(N_EXP_, 1, D_MODEL_))

    yp = _sc_gather_y(ys, dest)

    out = pl.pallas_call(
        _combine_kernel,
        in_specs=[
            pl.BlockSpec((N_ASSIGN_, D_MODEL_), lambda: (0, 0)),
            pl.BlockSpec((N_TOK_, LANES_), lambda: (0, 0)),
        ],
        out_specs=pl.BlockSpec((N_TOK_, D_MODEL_), lambda: (0, 0)),
        out_shape=jax.ShapeDtypeStruct((N_TOK_, D_MODEL_), jnp.float32),
    )(yp, w_pair)
    return out.reshape(orig_shape)


# speedup vs baseline: 1.7389x; 1.0965x over previous
import functools

import jax
import jax.numpy as jnp
from jax import lax
from jax.experimental import pallas as pl
from jax.experimental.pallas import tpu as pltpu
from jax.experimental.pallas import tpu_sc as plsc

D_MODEL_ = 768
D_FF_ = 3072
N_EXP_ = 8
N_TOK_ = 2048
LANES_ = 128
N_ASSIGN_ = 2 * N_TOK_
BLK_ = 512
GMAX_ = 15
S_PAD_ = GMAX_ * BLK_
CHUNK_ = 1024
N_CHUNK_ = N_ASSIGN_ // CHUNK_
SQRT1_2_ = 0.7071067811865476


def _gelu_exact(h):
    return h * 0.5 * (1.0 + lax.erf(h * jnp.float32(SQRT1_2_)))


def _route_meta_kernel(xt_ref, wr_ref, br_ref,
                       dest_ref, w_ref, eblk_ref, valid_ref, m_ref):
    lane = lax.broadcasted_iota(jnp.int32, (N_TOK_, LANES_), 1)
    evalid = lane < N_EXP_
    logits = lax.dot_general(
        xt_ref[...], wr_ref[...], (((1,), (0,)), ((), ())),
        preferred_element_type=jnp.float32) + br_ref[...]
    logits = jnp.where(evalid, logits, jnp.float32(-1e30))
    mx = jnp.max(logits, axis=1, keepdims=True)
    ex = jnp.where(evalid, jnp.exp(logits - mx), 0.0)
    probs = ex / jnp.sum(ex, axis=1, keepdims=True)
    m1 = jnp.max(probs, axis=1, keepdims=True)
    i1 = jnp.min(jnp.where(probs == m1, lane, N_EXP_), axis=1, keepdims=True)
    sel1 = lane == i1
    probs2 = jnp.where(sel1, -1.0, probs)
    m2 = jnp.max(probs2, axis=1, keepdims=True)
    i2 = jnp.min(jnp.where(probs2 == m2, lane, N_EXP_), axis=1, keepdims=True)
    sel2 = lane == i2
    tot = jnp.clip(m1 + m2, 1e-9, None)
    w_ref[...] = (jnp.where(lane == 0, m1, 0.0)
                  + jnp.where(lane == 1, m2, 0.0)) / tot

    sel1f = sel1.astype(jnp.float32)
    sel2f = sel2.astype(jnp.float32)
    m_ref[:N_TOK_] = sel1f
    m_ref[N_TOK_:] = sel2f
    counts = (jnp.sum(sel1f, axis=0, keepdims=True)
              + jnp.sum(sel2f, axis=0, keepdims=True))

    nblk = jnp.floor((counts + (BLK_ - 1)) * (1.0 / BLK_))
    rowi = lax.broadcasted_iota(jnp.int32, (LANES_, LANES_), 0)
    coli = lax.broadcasted_iota(jnp.int32, (LANES_, LANES_), 1)
    triu_strict = (rowi < coli).astype(jnp.float32)
    blkoff = lax.dot_general(nblk, triu_strict, (((1,), (0,)), ((), ())),
                             preferred_element_type=jnp.float32)
    off_pad = blkoff * jnp.float32(BLK_)

    row = lax.broadcasted_iota(jnp.int32, (CHUNK_, CHUNK_), 0)
    col = lax.broadcasted_iota(jnp.int32, (CHUNK_, CHUNK_), 1)
    tril_strict = (col < row).astype(jnp.float32)

    def body(c, carry):
        off = pl.multiple_of(c * CHUNK_, CHUNK_)
        mc = m_ref[pl.ds(off, CHUNK_), :]
        rank_c = lax.dot_general(
            tril_strict, mc, (((1,), (0,)), ((), ())),
            preferred_element_type=jnp.float32) + carry
        dest_ref[pl.ds(off, CHUNK_), :] = jnp.sum(
            (rank_c + off_pad) * mc, axis=1, keepdims=True).astype(jnp.int32)
        return carry + jnp.sum(mc, axis=0, keepdims=True)

    lax.fori_loop(0, N_CHUNK_, body, jnp.zeros((1, LANES_), jnp.float32))

    ones = jnp.ones((LANES_, LANES_), jnp.float32)
    onehot8 = jnp.logical_and(rowi == coli, rowi < N_EXP_).astype(jnp.float32)
    off_rows = onehot8 * blkoff
    off_bcast = lax.dot_general(off_rows, ones, (((1,), (0,)), ((), ())),
                                preferred_element_type=jnp.float32)
    g_row = lax.broadcasted_iota(jnp.int32, (N_EXP_, LANES_), 1
                                 ).astype(jnp.float32)
    ge = (g_row >= off_bcast[:N_EXP_]).astype(jnp.float32)
    eblk_ref[...] = (jnp.sum(ge, axis=0, keepdims=True) - 1.0
                     ).astype(jnp.int32)
    emask = lax.broadcasted_iota(jnp.int32, (1, LANES_), 1) < N_EXP_
    total = lax.dot_general(jnp.where(emask, nblk, 0.0), ones,
                            (((1,), (0,)), ((), ())),
                            preferred_element_type=jnp.float32)
    gl = lax.broadcasted_iota(jnp.int32, (1, LANES_), 1).astype(jnp.float32)
    valid_ref[...] = (gl < total).astype(jnp.int32)


def _gmm_kernel(eblk_ref, valid_ref, xs_ref, w1_ref, b1_ref, w2_ref, b2_ref,
                ys_ref):
    g = pl.program_id(0)

    @pl.when(valid_ref[g] == 1)
    def _():
        h = lax.dot_general(
            xs_ref[...], w1_ref[0], (((1,), (0,)), ((), ())),
            preferred_element_type=jnp.float32) + b1_ref[0]
        h = _gelu_exact(h)
        ys_ref[...] = lax.dot_general(
            h, w2_ref[0], (((1,), (0,)), ((), ())),
            preferred_element_type=jnp.float32) + b2_ref[0]


def _combine_kernel(yp_ref, w_ref, out_ref):
    lane = lax.broadcasted_iota(jnp.int32, (N_TOK_, LANES_), 1)
    w0 = jnp.sum(jnp.where(lane == 0, w_ref[...], 0.0), axis=1, keepdims=True)
    w1 = jnp.sum(jnp.where(lane == 1, w_ref[...], 0.0), axis=1, keepdims=True)
    out_ref[...] = w0 * yp_ref[:N_TOK_] + w1 * yp_ref[N_TOK_:]


_NC_ = 2
_NS_ = 16
_NW_ = _NC_ * _NS_
_APW_ = N_ASSIGN_ // _NW_


def _sc_mesh():
    return plsc.VectorSubcoreMesh(core_axis_name="c", subcore_axis_name="s",
                                  num_cores=_NC_, num_subcores=_NS_)


def _scatter_x_body(xt_hbm, dest_hbm, xs_hbm, idx_v, rows_v, sem):
    wid = lax.axis_index("s") * _NC_ + lax.axis_index("c")
    base = wid * _APW_
    src = lax.rem(base, N_TOK_)
    pltpu.sync_copy(dest_hbm.at[pl.ds(base, _APW_)], idx_v)
    pltpu.sync_copy(xt_hbm.at[pl.ds(src, _APW_)], rows_v)
    pltpu.async_copy(rows_v, xs_hbm.at[idx_v], sem).wait()


def _gather_y_body(ys_hbm, dest_hbm, yp_hbm, idx_v, rows_v, sem):
    wid = lax.axis_index("s") * _NC_ + lax.axis_index("c")
    base = wid * _APW_
    pltpu.sync_copy(dest_hbm.at[pl.ds(base, _APW_)], idx_v)
    pltpu.async_copy(ys_hbm.at[idx_v], rows_v, sem).wait()
    pltpu.sync_copy(rows_v, yp_hbm.at[pl.ds(base, _APW_)])


def _sc_scatter_x(xt, dest):
    return pl.kernel(
        _scatter_x_body,
        out_type=jax.ShapeDtypeStruct((S_PAD_, D_MODEL_), jnp.float32),
        mesh=_sc_mesh(),
        scratch_types=[
            pltpu.VMEM((_APW_,), jnp.int32),
            pltpu.VMEM((_APW_, D_MODEL_), jnp.float32),
            pltpu.SemaphoreType.DMA,
        ],
    )(xt, dest)


def _sc_gather_y(ys, dest):
    return pl.kernel(
        _gather_y_body,
        out_type=jax.ShapeDtypeStruct((N_ASSIGN_, D_MODEL_), jnp.float32),
        mesh=_sc_mesh(),
        scratch_types=[
            pltpu.VMEM((_APW_,), jnp.int32),
            pltpu.VMEM((_APW_, D_MODEL_), jnp.float32),
            pltpu.SemaphoreType.DMA,
        ],
    )(ys, dest)


@jax.jit
def kernel(x, Wr, br, W1, b1, W2, b2):
    orig_shape = x.shape
    xt = x.reshape(-1, D_MODEL_)
    wr_pad = jnp.zeros((D_MODEL_, LANES_), jnp.float32).at[:, :N_EXP_].set(Wr)
    br_pad = jnp.zeros((LANES_,), jnp.float32).at[:N_EXP_].set(br)

    dest2d, w_pair, eblk2d, valid2d = pl.pallas_call(
        _route_meta_kernel,
        in_specs=[
            pl.BlockSpec((N_TOK_, D_MODEL_), lambda: (0, 0)),
            pl.BlockSpec((D_MODEL_, LANES_), lambda: (0, 0)),
            pl.BlockSpec((LANES_,), lambda: (0,)),
        ],
        out_specs=[
            pl.BlockSpec((N_ASSIGN_, 1), lambda: (0, 0)),
            pl.BlockSpec((N_TOK_, LANES_), lambda: (0, 0)),
            pl.BlockSpec((1, LANES_), lambda: (0, 0)),
            pl.BlockSpec((1, LANES_), lambda: (0, 0)),
        ],
        out_shape=[
            jax.ShapeDtypeStruct((N_ASSIGN_, 1), jnp.int32),
            jax.ShapeDtypeStruct((N_TOK_, LANES_), jnp.float32),
            jax.ShapeDtypeStruct((1, LANES_), jnp.int32),
            jax.ShapeDtypeStruct((1, LANES_), jnp.int32),
        ],
        scratch_shapes=[
            pltpu.VMEM((N_ASSIGN_, LANES_), jnp.float32),
        ],
    )(xt, wr_pad, br_pad)

    dest = dest2d.reshape(N_ASSIGN_)
    eblk = eblk2d.reshape(LANES_)[:GMAX_]
    valid = valid2d.reshape(LANES_)[:GMAX_]

    xs = _sc_scatter_x(xt, dest)

    ys = pl.pallas_call(
        _gmm_kernel,
        grid_spec=pltpu.PrefetchScalarGridSpec(
            num_scalar_prefetch=2,
            grid=(GMAX_,),
            in_specs=[
                pl.BlockSpec((BLK_, D_MODEL_), lambda g, eb, vd: (g, 0)),
                pl.BlockSpec((1, D_MODEL_, D_FF_),
                             lambda g, eb, vd: (eb[g], 0, 0)),
                pl.BlockSpec((1, 1, D_FF_), lambda g, eb, vd: (eb[g], 0, 0)),
                pl.BlockSpec((1, D_FF_, D_MODEL_),
                             lambda g, eb, vd: (eb[g], 0, 0)),
                pl.BlockSpec((1, 1, D_MODEL_),
                             lambda g, eb, vd: (eb[g], 0, 0)),
            ],
            out_specs=pl.BlockSpec((BLK_, D_MODEL_), lambda g, eb, vd: (g, 0)),
        ),
        out_shape=jax.ShapeDtypeStruct((S_PAD_, D_MODEL_), jnp.float32),
    )(eblk, valid, xs, W1, b1.reshape(N_EXP_, 1, D_FF_), W2,
      b2.reshape(N_EXP_, 1, D_MODEL_))

    yp = _sc_gather_y(ys, dest)

    out = pl.pallas_call(
        _combine_kernel,
        in_specs=[
            pl.BlockSpec((N_ASSIGN_, D_MODEL_), lambda: (0, 0)),
            pl.BlockSpec((N_TOK_, LANES_), lambda: (0, 0)),
        ],
        out_specs=pl.BlockSpec((N_TOK_, D_MODEL_), lambda: (0, 0)),
        out_shape=jax.ShapeDtypeStruct((N_TOK_, D_MODEL_), jnp.float32),
    )(yp, w_pair)
    return out.reshape(orig_shape)
